# baseline (speedup 1.0000x reference)
import functools

import jax
import jax.numpy as jnp
from jax import lax
from jax.experimental import pallas as pl
from jax.experimental.pallas import tpu as pltpu
from jax.experimental.pallas import tpu_sc as plsc

NC = 2
NS = 16
NW = NC * NS

N = 10000
E = 320000
D = 128
AUG = D + 16
CA = 125
NBA = E // NW // CA
NZ = N // NS
EW = E // NW
CR = 80
NBC = EW // CR

_MESH = plsc.VectorSubcoreMesh(core_axis_name="c", subcore_axis_name="s")
_SC_PARAMS = pltpu.CompilerParams(use_tc_tiling_on_sc=False)
_SC_PARAMS_TILED = pltpu.CompilerParams(use_tc_tiling_on_sc=True)


def _phase_a_body(xaug_hbm, src_hbm, dst_hbm, zeros_hbm, part_hbm,
                  src_r, dst_r, buf, acc, gsem, ssem, isem):
  c = lax.axis_index("c")
  s = lax.axis_index("s")
  wid = s * NC + c

  pltpu.sync_copy(zeros_hbm, buf.at[0])
  for k in range(NZ // CA):
    pltpu.sync_copy(buf.at[0], acc.at[pl.ds(s * NZ + k * CA, CA)])

  def idx_load(blk, start):
    sl = lax.rem(blk, 4)
    f = pltpu.async_copy if start else pltpu.make_async_copy
    ds_ = f(src_hbm.at[wid * NBA + blk], src_r.at[sl], isem.at[sl])
    dd_ = f(dst_hbm.at[wid * NBA + blk], dst_r.at[sl], isem.at[sl])
    if not start:
      ds_.wait()
      dd_.wait()

  def gather(blk, p, start):
    f = pltpu.async_copy if start else pltpu.make_async_copy
    d = f(xaug_hbm.at[src_r.at[lax.rem(blk, 4)]], buf.at[p], gsem.at[p])
    if not start:
      d.wait()

  def scat(blk, p, start):
    if start:
      pltpu.async_copy(buf.at[p], acc.at[dst_r.at[lax.rem(blk, 4)]],
                       ssem.at[p], add=True)
    else:
      pltpu.make_async_copy(buf.at[p], acc.at[dst_r.at[lax.rem(blk, 4)]],
                            ssem.at[p]).wait()

  idx_load(0, start=True)
  idx_load(0, start=False)
  idx_load(1, start=True)
  plsc.subcore_barrier()
  gather(0, 0, start=True)

  @pl.loop(0, NBA)
  def _(blk):
    p = lax.rem(blk, 2)
    q = 1 - p

    @pl.when(blk + 1 < NBA)
    def _():
      @pl.when(blk >= 1)
      def _():
        scat(blk - 1, q, start=False)
      idx_load(blk + 1, start=False)

      @pl.when(blk + 2 < NBA)
      def _():
        idx_load(blk + 2, start=True)
      gather(blk + 1, q, start=True)

    gather(blk, p, start=False)
    scat(blk, p, start=True)

  scat(NBA - 2, (NBA - 2) % 2, start=False)
  scat(NBA - 1, (NBA - 1) % 2, start=False)
  plsc.subcore_barrier()

  for k in range(NZ // CA):
    pltpu.sync_copy(acc.at[pl.ds(s * NZ + k * CA, CA)], buf.at[k % 2])
    pltpu.sync_copy(buf.at[k % 2], part_hbm.at[c, pl.ds(s * NZ + k * CA, CA)])


_phase_a = pl.kernel(
    _phase_a_body,
    out_type=jax.ShapeDtypeStruct((NC, N, AUG), jnp.float32),
    mesh=_MESH,
    scratch_types=[
        pltpu.VMEM((4, CA), jnp.int32),
        pltpu.VMEM((4, CA), jnp.int32),
        pltpu.VMEM((2, CA, AUG), jnp.float32),
        pltpu.VMEM_SHARED((N, AUG), jnp.float32),
        pltpu.SemaphoreType.DMA((2,)),
        pltpu.SemaphoreType.DMA((2,)),
        pltpu.SemaphoreType.DMA((4,)),
    ],
    compiler_params=_SC_PARAMS,
)


def _phase_b_block(x_ref, p_ref, ws_ref, wn_ref, b_ref, h_ref):
  ssum = p_ref[0] + p_ref[1]
  deg = jnp.maximum(ssum[:, D:D + 1], 1.0)
  agg = ssum[:, :D] / deg
  h = (jnp.dot(x_ref[...], ws_ref[...], preferred_element_type=jnp.float32)
       + jnp.dot(agg, wn_ref[...], preferred_element_type=jnp.float32)
       + b_ref[...])
  h_ref[...] = jnp.maximum(h, 0.0)


_BLK = 2000


def _phase_b(x, part, w_self, w_neigh, b2):
  return pl.pallas_call(
      _phase_b_block,
      grid=(N // _BLK,),
      in_specs=[
          pl.BlockSpec((_BLK, D), lambda i: (i, 0)),
          pl.BlockSpec((NC, _BLK, AUG), lambda i: (0, i, 0)),
          pl.BlockSpec((D, D), lambda i: (0, 0)),
          pl.BlockSpec((D, D), lambda i: (0, 0)),
          pl.BlockSpec((1, D), lambda i: (0, 0)),
      ],
      out_specs=pl.BlockSpec((_BLK, D), lambda i: (i, 0)),
      out_shape=jax.ShapeDtypeStruct((N, D), jnp.float32),
  )(x, part, w_self, w_neigh, b2)


def _phase_c_body(h_hbm, si_hbm, di_hbm, out_hbm,
                  si_v, di_v, bufs, bufd, gss, gsd, wss, wsd):
  c = lax.axis_index("c")
  s = lax.axis_index("s")
  wid = s * NC + c
  base = wid * EW
  pltpu.sync_copy(si_hbm.at[pl.ds(base, EW)], si_v)
  pltpu.sync_copy(di_hbm.at[pl.ds(base, EW)], di_v)

  def gather(blk, start):
    p = lax.rem(blk, 4)
    rows = pl.ds(blk * CR, CR)
    f = pltpu.async_copy if start else pltpu.make_async_copy
    ds_ = f(h_hbm.at[si_v.at[rows]], bufs.at[p], gss.at[p])
    dd_ = f(h_hbm.at[di_v.at[rows]], bufd.at[p], gsd.at[p])
    if not start:
      ds_.wait()
      dd_.wait()

  def write(blk, start):
    p = lax.rem(blk, 4)
    rows = pl.ds(base + blk * CR, CR)
    f = pltpu.async_copy if start else pltpu.make_async_copy
    ds_ = f(bufs.at[p], out_hbm.at[rows, pl.ds(0, D)], wss.at[p])
    dd_ = f(bufd.at[p], out_hbm.at[rows, pl.ds(D, D)], wsd.at[p])
    if not start:
      ds_.wait()
      dd_.wait()

  gather(0, start=True)
  gather(1, start=True)

  @pl.loop(0, NBC)
  def _(blk):
    @pl.when(blk + 2 < NBC)
    def _():
      @pl.when(blk >= 2)
      def _():
        write(blk - 2, start=False)
      gather(blk + 2, start=True)

    gather(blk, start=False)
    write(blk, start=True)

  write(NBC - 4, start=False)
  write(NBC - 3, start=False)
  write(NBC - 2, start=False)
  write(NBC - 1, start=False)


_phase_c = pl.kernel(
    _phase_c_body,
    out_type=jax.ShapeDtypeStruct((E, 2 * D), jnp.float32),
    mesh=_MESH,
    scratch_types=[
        pltpu.VMEM((EW,), jnp.int32),
        pltpu.VMEM((EW,), jnp.int32),
        pltpu.VMEM((4, CR, D), jnp.float32),
        pltpu.VMEM((4, CR, D), jnp.float32),
        pltpu.SemaphoreType.DMA((4,)),
        pltpu.SemaphoreType.DMA((4,)),
        pltpu.SemaphoreType.DMA((4,)),
        pltpu.SemaphoreType.DMA((4,)),
    ],
    compiler_params=_SC_PARAMS_TILED,
)


def kernel(x, edge_index, W_self, W_neigh, b):
  x = x.astype(jnp.float32)
  ei = edge_index.astype(jnp.int32)
  xaug = jnp.concatenate([x, jnp.ones((N, AUG - D), jnp.float32)], axis=1)
  zeros = jnp.zeros((CA, AUG), jnp.float32)
  src2 = ei[0].reshape(NW * NBA, CA)
  dst2 = ei[1].reshape(NW * NBA, CA)
  part = _phase_a(xaug, src2, dst2, zeros)
  h = _phase_b(x, part, W_self, W_neigh, b.reshape(1, D))
  return _phase_c(h, ei[0], ei[1])


# speedup vs baseline: 1.0006x; 1.0006x over previous
import functools

import jax
import jax.numpy as jnp
from jax import lax
from jax.experimental import pallas as pl
from jax.experimental.pallas import tpu as pltpu
from jax.experimental.pallas import tpu_sc as plsc

NC = 2
NS = 16
NW = NC * NS

N = 10000
E = 320000
D = 128
AUG = D + 16
CA = 125
NBA = E // NW // CA
NZ = N // NS
EW = E // NW
CR = 80
NBC = EW // CR

_MESH = plsc.VectorSubcoreMesh(core_axis_name="c", subcore_axis_name="s")
_SC_PARAMS = pltpu.CompilerParams(use_tc_tiling_on_sc=False)
_SC_PARAMS_TILED = pltpu.CompilerParams(use_tc_tiling_on_sc=True)


def _phase_a_body(xaug_hbm, src_hbm, dst_hbm, zeros_hbm, part_hbm,
                  src_r, dst_r, buf, acc, gsem, ssem, isem):
  c = lax.axis_index("c")
  s = lax.axis_index("s")
  wid = s * NC + c

  pltpu.sync_copy(zeros_hbm, buf.at[0])
  for k in range(NZ // CA):
    pltpu.sync_copy(buf.at[0], acc.at[pl.ds(s * NZ + k * CA, CA)])

  def idx_load(blk, start):
    sl = lax.rem(blk, 4)
    f = pltpu.async_copy if start else pltpu.make_async_copy
    ds_ = f(src_hbm.at[wid * NBA + blk], src_r.at[sl], isem.at[sl])
    dd_ = f(dst_hbm.at[wid * NBA + blk], dst_r.at[sl], isem.at[sl])
    if not start:
      ds_.wait()
      dd_.wait()

  def gather(blk, p, start):
    f = pltpu.async_copy if start else pltpu.make_async_copy
    d = f(xaug_hbm.at[src_r.at[lax.rem(blk, 4)]], buf.at[p], gsem.at[p])
    if not start:
      d.wait()

  def scat(blk, p, start):
    if start:
      pltpu.async_copy(buf.at[p], acc.at[dst_r.at[lax.rem(blk, 4)]],
                       ssem.at[p], add=True)
    else:
      pltpu.make_async_copy(buf.at[p], acc.at[dst_r.at[lax.rem(blk, 4)]],
                            ssem.at[p]).wait()

  idx_load(0, start=True)
  idx_load(0, start=False)
  idx_load(1, start=True)
  plsc.subcore_barrier()
  gather(0, 0, start=True)

  @pl.loop(0, NBA)
  def _(blk):
    p = lax.rem(blk, 2)
    q = 1 - p

    @pl.when(blk + 1 < NBA)
    def _():
      @pl.when(blk >= 1)
      def _():
        scat(blk - 1, q, start=False)
      idx_load(blk + 1, start=False)

      @pl.when(blk + 2 < NBA)
      def _():
        idx_load(blk + 2, start=True)
      gather(blk + 1, q, start=True)

    gather(blk, p, start=False)
    scat(blk, p, start=True)

  scat(NBA - 2, (NBA - 2) % 2, start=False)
  scat(NBA - 1, (NBA - 1) % 2, start=False)
  plsc.subcore_barrier()

  for k in range(NZ // CA):
    pltpu.sync_copy(acc.at[pl.ds(s * NZ + k * CA, CA)], buf.at[k % 2])
    pltpu.sync_copy(buf.at[k % 2], part_hbm.at[c, pl.ds(s * NZ + k * CA, CA)])


_phase_a = pl.kernel(
    _phase_a_body,
    out_type=jax.ShapeDtypeStruct((NC, N, AUG), jnp.float32),
    mesh=_MESH,
    scratch_types=[
        pltpu.VMEM((4, CA), jnp.int32),
        pltpu.VMEM((4, CA), jnp.int32),
        pltpu.VMEM((2, CA, AUG), jnp.float32),
        pltpu.VMEM_SHARED((N, AUG), jnp.float32),
        pltpu.SemaphoreType.DMA((2,)),
        pltpu.SemaphoreType.DMA((2,)),
        pltpu.SemaphoreType.DMA((4,)),
    ],
    compiler_params=_SC_PARAMS,
)


def _phase_b_block(x_ref, p_ref, ws_ref, wn_ref, b_ref, h_ref):
  ssum = p_ref[0] + p_ref[1]
  deg = jnp.maximum(ssum[:, D:D + 1], 1.0)
  agg = ssum[:, :D] / deg
  h = (jnp.dot(x_ref[...], ws_ref[...], preferred_element_type=jnp.float32)
       + jnp.dot(agg, wn_ref[...], preferred_element_type=jnp.float32)
       + b_ref[...])
  h_ref[...] = jnp.maximum(h, 0.0)


_BLK = 2000


def _phase_b(x, part, w_self, w_neigh, b2):
  return pl.pallas_call(
      _phase_b_block,
      grid=(N // _BLK,),
      in_specs=[
          pl.BlockSpec((_BLK, D), lambda i: (i, 0)),
          pl.BlockSpec((NC, _BLK, AUG), lambda i: (0, i, 0)),
          pl.BlockSpec((D, D), lambda i: (0, 0)),
          pl.BlockSpec((D, D), lambda i: (0, 0)),
          pl.BlockSpec((1, D), lambda i: (0, 0)),
      ],
      out_specs=pl.BlockSpec((_BLK, D), lambda i: (i, 0)),
      out_shape=jax.ShapeDtypeStruct((N, D), jnp.float32),
  )(x, part, w_self, w_neigh, b2)


def _phase_c_body(h_hbm, si_hbm, di_hbm, out_hbm,
                  si_v, di_v, bufs, bufd, gss, gsd, wss, wsd):
  c = lax.axis_index("c")
  s = lax.axis_index("s")
  wid = s * NC + c
  base = wid * EW
  pltpu.sync_copy(si_hbm.at[pl.ds(base, EW)], si_v)
  pltpu.sync_copy(di_hbm.at[pl.ds(base, EW)], di_v)

  def gather(blk, start):
    p = lax.rem(blk, 5)
    rows = pl.ds(blk * CR, CR)
    f = pltpu.async_copy if start else pltpu.make_async_copy
    ds_ = f(h_hbm.at[si_v.at[rows]], bufs.at[p], gss.at[p])
    dd_ = f(h_hbm.at[di_v.at[rows]], bufd.at[p], gsd.at[p])
    if not start:
      ds_.wait()
      dd_.wait()

  def write(blk, start):
    p = lax.rem(blk, 5)
    rows = pl.ds(base + blk * CR, CR)
    f = pltpu.async_copy if start else pltpu.make_async_copy
    ds_ = f(bufs.at[p], out_hbm.at[rows, pl.ds(0, D)], wss.at[p])
    dd_ = f(bufd.at[p], out_hbm.at[rows, pl.ds(D, D)], wsd.at[p])
    if not start:
      ds_.wait()
      dd_.wait()

  gather(0, start=True)
  gather(1, start=True)

  @pl.loop(0, NBC)
  def _(blk):
    @pl.when(blk + 2 < NBC)
    def _():
      @pl.when(blk >= 3)
      def _():
        write(blk - 3, start=False)
      gather(blk + 2, start=True)

    gather(blk, start=False)
    write(blk, start=True)

  write(NBC - 5, start=False)
  write(NBC - 4, start=False)
  write(NBC - 3, start=False)
  write(NBC - 2, start=False)
  write(NBC - 1, start=False)


_phase_c = pl.kernel(
    _phase_c_body,
    out_type=jax.ShapeDtypeStruct((E, 2 * D), jnp.float32),
    mesh=_MESH,
    scratch_types=[
        pltpu.VMEM((EW,), jnp.int32),
        pltpu.VMEM((EW,), jnp.int32),
        pltpu.VMEM((5, CR, D), jnp.float32),
        pltpu.VMEM((5, CR, D), jnp.float32),
        pltpu.SemaphoreType.DMA((5,)),
        pltpu.SemaphoreType.DMA((5,)),
        pltpu.SemaphoreType.DMA((5,)),
        pltpu.SemaphoreType.DMA((5,)),
    ],
    compiler_params=_SC_PARAMS_TILED,
)


def kernel(x, edge_index, W_self, W_neigh, b):
  x = x.astype(jnp.float32)
  ei = edge_index.astype(jnp.int32)
  xaug = jnp.concatenate([x, jnp.ones((N, AUG - D), jnp.float32)], axis=1)
  zeros = jnp.zeros((CA, AUG), jnp.float32)
  src2 = ei[0].reshape(NW * NBA, CA)
  dst2 = ei[1].reshape(NW * NBA, CA)
  part = _phase_a(xaug, src2, dst2, zeros)
  h = _phase_b(x, part, W_self, W_neigh, b.reshape(1, D))
  return _phase_c(h, ei[0], ei[1])
